# Initial kernel scaffold; baseline (speedup 1.0000x reference)
import jax
import jax.numpy as jnp
from jax.experimental import pallas as pl


def kernel(x, in_proj_w, in_proj_b, out_proj_w, out_proj_b, ln1_g, ln1_b, ln2_g, ln2_b, wg, w1, b1, w2, b2):
    raise NotImplementedError("write your pallas kernel here")


# speedup vs baseline: 1.2226x; 1.2226x over previous
import functools

import jax
import jax.numpy as jnp
from jax import lax
from jax.experimental import pallas as pl
from jax.experimental.pallas import tpu as pltpu
from jax.experimental.pallas import tpu_sc as plsc

F32 = jnp.float32
I32 = jnp.int32

S, H, NH, E, FF = 2048, 1024, 16, 8, 4096
DH = H // NH
CAP = int(1.25 * S * 2 / E)
NSLOT = E * CAP
PADROW = S
DUMMY = NSLOT
EPAD = 128

NC, NSUB = 2, 16
NW = NC * NSUB


def _qkv_body(x_ref, w_ref, b_ref, o_ref):
    o_ref[...] = (
        jnp.dot(x_ref[...], w_ref[...], preferred_element_type=F32) + b_ref[...]
    )


def _qkv(x2d, wi_t, bi2d):
    return pl.pallas_call(
        _qkv_body,
        grid=(8, 6),
        in_specs=[
            pl.BlockSpec((S // 8, H), lambda i, j: (i, 0)),
            pl.BlockSpec((H, 512), lambda i, j: (0, j)),
            pl.BlockSpec((1, 512), lambda i, j: (0, j)),
        ],
        out_specs=pl.BlockSpec((S // 8, 512), lambda i, j: (i, j)),
        out_shape=jax.ShapeDtypeStruct((S, 3 * H), F32),
    )(x2d, wi_t, bi2d)


def _attn_body(q_ref, k_ref, v_ref, o_ref):
    q = q_ref[0]
    s = lax.dot_general(
        q, k_ref[0], (((1,), (1,)), ((), ())), preferred_element_type=F32
    ) * (1.0 / 8.0)
    m = jnp.max(s, axis=-1, keepdims=True)
    p = jnp.exp(s - m)
    p = p / jnp.sum(p, axis=-1, keepdims=True)
    o_ref[0] = jnp.dot(p, v_ref[0], preferred_element_type=F32)


def _attention(q3, k3, v3):
    qb = S // 8
    return pl.pallas_call(
        _attn_body,
        grid=(NH, 8),
        in_specs=[
            pl.BlockSpec((1, qb, DH), lambda h, i: (h, i, 0)),
            pl.BlockSpec((1, S, DH), lambda h, i: (h, 0, 0)),
            pl.BlockSpec((1, S, DH), lambda h, i: (h, 0, 0)),
        ],
        out_specs=pl.BlockSpec((1, qb, DH), lambda h, i: (h, i, 0)),
        out_shape=jax.ShapeDtypeStruct((NH, S, DH), F32),
    )(q3, k3, v3)


def _ln1_body(a_ref, wo_ref, bo_ref, x_ref, g_ref, b_ref, wg_ref, x1_ref, lg_ref):
    y = (
        jnp.dot(a_ref[...], wo_ref[...], preferred_element_type=F32)
        + bo_ref[...]
        + x_ref[...]
    )
    mu = jnp.mean(y, axis=-1, keepdims=True)
    yc = y - mu
    var = jnp.mean(yc * yc, axis=-1, keepdims=True)
    x1 = yc / jnp.sqrt(var + 1e-5) * g_ref[...] + b_ref[...]
    x1_ref[...] = x1
    lg_ref[...] = jnp.dot(x1, wg_ref[...], preferred_element_type=F32)


def _ln1(attn, wo_t, bo2d, x2d, g2d, b2d, wg_pad):
    rb = S // 8
    return pl.pallas_call(
        _ln1_body,
        grid=(8,),
        in_specs=[
            pl.BlockSpec((rb, H), lambda i: (i, 0)),
            pl.BlockSpec((H, H), lambda i: (0, 0)),
            pl.BlockSpec((1, H), lambda i: (0, 0)),
            pl.BlockSpec((rb, H), lambda i: (i, 0)),
            pl.BlockSpec((1, H), lambda i: (0, 0)),
            pl.BlockSpec((1, H), lambda i: (0, 0)),
            pl.BlockSpec((H, EPAD), lambda i: (0, 0)),
        ],
        out_specs=[
            pl.BlockSpec((rb, H), lambda i: (i, 0)),
            pl.BlockSpec((rb, EPAD), lambda i: (i, 0)),
        ],
        out_shape=[
            jax.ShapeDtypeStruct((S, H), F32),
            jax.ShapeDtypeStruct((S, EPAD), F32),
        ],
    )(attn, wo_t, bo2d, x2d, g2d, b2d, wg_pad)


_RB = 256
_NB = S // _RB


def _router_body(lg_ref, s1_ref, s2_ref, g1_ref, g2_ref,
                 m1_ref, m2_ref, p_ref, i1_ref, i2_ref, bs1_ref, bs2_ref):
    eiota = lax.broadcasted_iota(I32, (_RB, EPAD), 1)
    neg = jnp.float32(-1e30)
    for i in range(_NB):
        rows = pl.ds(i * _RB, _RB)
        logits = jnp.where(eiota < E, lg_ref[rows, :], neg)
        mx = jnp.max(logits, axis=-1, keepdims=True)
        ex = jnp.exp(logits - mx)
        probs = ex / jnp.sum(ex, axis=-1, keepdims=True)
        m1v = jnp.max(probs, axis=-1, keepdims=True)
        idx1 = jnp.min(jnp.where(probs == m1v, eiota, EPAD), axis=-1, keepdims=True)
        mask1 = (eiota == idx1).astype(F32)
        probs2 = probs * (1.0 - mask1)
        m2v = jnp.max(probs2, axis=-1, keepdims=True)
        idx2 = jnp.min(jnp.where(probs2 == m2v, eiota, EPAD), axis=-1, keepdims=True)
        mask2 = (eiota == idx2).astype(F32)
        m1_ref[rows, :] = mask1
        m2_ref[rows, :] = mask2
        p_ref[rows, :] = probs
        i1_ref[rows, :] = idx1
        i2_ref[rows, :] = idx2
        bs1_ref[pl.ds(i, 1), :] = jnp.sum(mask1, axis=0, keepdims=True)
        bs2_ref[pl.ds(i, 1), :] = jnp.sum(mask2, axis=0, keepdims=True)
    bs1 = bs1_ref[...]
    bs2 = bs2_ref[...]
    count1 = jnp.sum(bs1, axis=0, keepdims=True)
    tri = (
        lax.broadcasted_iota(I32, (_RB, _RB), 1)
        < lax.broadcasted_iota(I32, (_RB, _RB), 0)
    ).astype(F32)
    run1 = jnp.zeros((1, EPAD), F32)
    run2 = jnp.zeros((1, EPAD), F32)
    capf = jnp.float32(CAP)
    for i in range(_NB):
        rows = pl.ds(i * _RB, _RB)
        m1 = m1_ref[rows, :]
        m2 = m2_ref[rows, :]
        probs = p_ref[rows, :]
        loc1 = jnp.dot(tri, m1, preferred_element_type=F32) + run1
        loc2 = jnp.dot(tri, m2, preferred_element_type=F32) + run2 + count1
        run1 = run1 + bs1[i : i + 1, :]
        run2 = run2 + bs2[i : i + 1, :]
        m1c = m1 * (loc1 < capf).astype(F32)
        m2c = m2 * (loc2 < capf).astype(F32)
        g1 = jnp.sum(probs * m1c, axis=-1, keepdims=True)
        g2 = jnp.sum(probs * m2c, axis=-1, keepdims=True)
        den = g1 + g2 + 1e-9
        pos1 = jnp.sum(loc1 * m1c, axis=-1, keepdims=True).astype(I32)
        pos2 = jnp.sum(loc2 * m2c, axis=-1, keepdims=True).astype(I32)
        kept1 = jnp.sum(m1c, axis=-1, keepdims=True) > 0.0
        kept2 = jnp.sum(m2c, axis=-1, keepdims=True) > 0.0
        s1_ref[rows, :] = jnp.where(kept1, i1_ref[rows, :] * CAP + pos1, DUMMY)
        s2_ref[rows, :] = jnp.where(kept2, i2_ref[rows, :] * CAP + pos2, DUMMY)
        g1_ref[rows, :] = g1 / den
        g2_ref[rows, :] = g2 / den


def _router(lg):
    return pl.pallas_call(
        _router_body,
        in_specs=[pl.BlockSpec((S, EPAD), lambda: (0, 0))],
        out_specs=[
            pl.BlockSpec((S, 1), lambda: (0, 0)),
            pl.BlockSpec((S, 1), lambda: (0, 0)),
            pl.BlockSpec((S, 1), lambda: (0, 0)),
            pl.BlockSpec((S, 1), lambda: (0, 0)),
        ],
        out_shape=[
            jax.ShapeDtypeStruct((S, 1), I32),
            jax.ShapeDtypeStruct((S, 1), I32),
            jax.ShapeDtypeStruct((S, 1), F32),
            jax.ShapeDtypeStruct((S, 1), F32),
        ],
        scratch_shapes=[
            pltpu.VMEM((S, EPAD), F32),
            pltpu.VMEM((S, EPAD), F32),
            pltpu.VMEM((S, EPAD), F32),
            pltpu.VMEM((S, 1), I32),
            pltpu.VMEM((S, 1), I32),
            pltpu.VMEM((_NB, EPAD), F32),
            pltpu.VMEM((_NB, EPAD), F32),
        ],
    )(lg)


SLOT_PAD = 5136
ROWS_CH = 40
SL_PER_W = NSLOT // NW


def _disp_body(s1_hbm, s2_hbm, g1_hbm, g2_hbm, x1p_hbm,
               ein_hbm, gate_hbm,
               s1_v, s2_v, g1_v, g2_v, stok_v, gate_v, rows_v, sem):
    c = lax.axis_index("c")
    sid = lax.axis_index("s")
    wid = sid * NC + c
    pltpu.sync_copy(s1_hbm, s1_v)
    pltpu.sync_copy(s2_hbm, s2_v)
    pltpu.sync_copy(g1_hbm, g1_v)
    pltpu.sync_copy(g2_hbm, g2_v)

    def init_body(j, _):
        stok_v[pl.ds(j * 16, 16)] = jnp.full((16,), PADROW, I32)
        gate_v[pl.ds(j * 16, 16)] = jnp.zeros((16,), F32)
        return ()

    lax.fori_loop(0, SLOT_PAD // 16, init_body, ())

    def scat_body(i, _):
        tok = lax.broadcasted_iota(I32, (16,), 0) + i * 16
        sv1 = s1_v[pl.ds(i * 16, 16)]
        sv2 = s2_v[pl.ds(i * 16, 16)]
        plsc.store_scatter(stok_v, [sv1], tok)
        plsc.store_scatter(stok_v, [sv2], tok)
        plsc.store_scatter(gate_v, [sv1], g1_v[pl.ds(i * 16, 16)])
        plsc.store_scatter(gate_v, [sv2], g2_v[pl.ds(i * 16, 16)])
        return ()

    lax.fori_loop(0, S // 16, scat_body, ())

    @pl.when(wid == 0)
    def _():
        pltpu.sync_copy(gate_v.at[pl.ds(0, NSLOT + 8)], gate_hbm)

    for ch in range(SL_PER_W // ROWS_CH):
        base = wid * SL_PER_W + ch * ROWS_CH
        cp = pltpu.async_copy(
            x1p_hbm.at[stok_v.at[pl.ds(base, ROWS_CH)]], rows_v, sem
        )
        cp.wait()
        pltpu.sync_copy(rows_v, ein_hbm.at[pl.ds(base, ROWS_CH)])


def _dispatch_sc(s1, s2, g1, g2, x1p):
    f = pl.kernel(
        _disp_body,
        out_type=[
            jax.ShapeDtypeStruct((NSLOT, H), F32),
            jax.ShapeDtypeStruct((NSLOT + 8,), F32),
        ],
        mesh=plsc.VectorSubcoreMesh(core_axis_name="c", subcore_axis_name="s"),
        compiler_params=pltpu.CompilerParams(needs_layout_passes=False),
        scratch_types=[
            pltpu.VMEM((S,), I32),
            pltpu.VMEM((S,), I32),
            pltpu.VMEM((S,), F32),
            pltpu.VMEM((S,), F32),
            pltpu.VMEM((SLOT_PAD,), I32),
            pltpu.VMEM((SLOT_PAD,), F32),
            pltpu.VMEM((ROWS_CH, H), F32),
            pltpu.SemaphoreType.DMA,
        ],
    )
    return f(s1, s2, g1, g2, x1p)


_FB = 1024
_SQRT1_2 = 0.7071067811865476


def _erf(x):
    a1, a2, a3, a4, a5 = (
        0.254829592, -0.284496736, 1.421413741, -1.453152027, 1.061405429,
    )
    p = 0.3275911
    ax = jnp.abs(x)
    t = 1.0 / (1.0 + p * ax)
    y = 1.0 - ((((a5 * t + a4) * t + a3) * t + a2) * t + a1) * t * jnp.exp(-ax * ax)
    return jnp.sign(x) * y


def _gelu(x):
    return 0.5 * x * (1.0 + _erf(x * _SQRT1_2))


def _ffn_body(xin_ref, w1_ref, b1_ref, w2_ref, b2_ref, g_ref, o_ref):
    f = pl.program_id(1)
    nf = pl.num_programs(1)
    h = _gelu(
        jnp.dot(xin_ref[...], w1_ref[0], preferred_element_type=F32) + b1_ref[0]
    )
    part = jnp.dot(h, w2_ref[0], preferred_element_type=F32)

    @pl.when(f == 0)
    def _():
        o_ref[...] = part

    @pl.when(f > 0)
    def _():
        o_ref[...] += part

    @pl.when(f == nf - 1)
    def _():
        o_ref[...] = (o_ref[...] + b2_ref[0]) * g_ref[0]


def _ffn(ein, w1, b1, w2, b2, gate3d):
    return pl.pallas_call(
        _ffn_body,
        grid=(E, FF // _FB),
        in_specs=[
            pl.BlockSpec((CAP, H), lambda e, f: (e, 0)),
            pl.BlockSpec((1, H, _FB), lambda e, f: (e, 0, f)),
            pl.BlockSpec((1, 1, _FB), lambda e, f: (e, 0, f)),
            pl.BlockSpec((1, _FB, H), lambda e, f: (e, f, 0)),
            pl.BlockSpec((1, 1, H), lambda e, f: (e, 0, 0)),
            pl.BlockSpec((1, CAP, 1), lambda e, f: (e, 0, 0)),
        ],
        out_specs=pl.BlockSpec((CAP, H), lambda e, f: (e, 0)),
        out_shape=jax.ShapeDtypeStruct((NSLOT, H), F32),
    )(ein, w1, b1.reshape(E, 1, FF), w2, b2.reshape(E, 1, H), gate3d)


TPB = S // NW
CHT = 32


def _comb_body(s1_hbm, s2_hbm, eop_hbm, out_hbm, i1_v, i2_v, r1_v, r2_v, sem1, sem2):
    c = lax.axis_index("c")
    sid = lax.axis_index("s")
    wid = sid * NC + c
    t0 = wid * TPB
    pltpu.sync_copy(s1_hbm.at[pl.ds(t0, TPB)], i1_v)
    pltpu.sync_copy(s2_hbm.at[pl.ds(t0, TPB)], i2_v)
    for ch in range(TPB // CHT):
        cb = ch * CHT
        cp1 = pltpu.async_copy(eop_hbm.at[i1_v.at[pl.ds(cb, CHT)]], r1_v, sem1)
        cp2 = pltpu.async_copy(eop_hbm.at[i2_v.at[pl.ds(cb, CHT)]], r2_v, sem2)
        cp1.wait()
        cp2.wait()

        def row_body(r, _):
            for cc in range(H // 16):
                sl = pl.ds(cc * 16, 16)
                r1_v[r, sl] = r1_v[r, sl] + r2_v[r, sl]
            return ()

        lax.fori_loop(0, CHT, row_body, ())
        pltpu.sync_copy(r1_v, out_hbm.at[pl.ds(t0 + cb, CHT)])


def _combine_sc(s1, s2, eop):
    f = pl.kernel(
        _comb_body,
        out_type=jax.ShapeDtypeStruct((S, H), F32),
        mesh=plsc.VectorSubcoreMesh(core_axis_name="c", subcore_axis_name="s"),
        compiler_params=pltpu.CompilerParams(needs_layout_passes=False),
        scratch_types=[
            pltpu.VMEM((TPB,), I32),
            pltpu.VMEM((TPB,), I32),
            pltpu.VMEM((CHT, H), F32),
            pltpu.VMEM((CHT, H), F32),
            pltpu.SemaphoreType.DMA,
            pltpu.SemaphoreType.DMA,
        ],
    )
    return f(s1, s2, eop)


def _ln2_body(x1_ref, m_ref, g_ref, b_ref, o_ref):
    y = x1_ref[...] + m_ref[...]
    mu = jnp.mean(y, axis=-1, keepdims=True)
    yc = y - mu
    var = jnp.mean(yc * yc, axis=-1, keepdims=True)
    o_ref[...] = yc * lax.rsqrt(var + 1e-5) * g_ref[...] + b_ref[...]


def _ln2(x1, moe, g2d, b2d):
    rb = S // 8
    return pl.pallas_call(
        _ln2_body,
        grid=(8,),
        in_specs=[
            pl.BlockSpec((rb, H), lambda i: (i, 0)),
            pl.BlockSpec((rb, H), lambda i: (i, 0)),
            pl.BlockSpec((1, H), lambda i: (0, 0)),
            pl.BlockSpec((1, H), lambda i: (0, 0)),
        ],
        out_specs=pl.BlockSpec((rb, H), lambda i: (i, 0)),
        out_shape=jax.ShapeDtypeStruct((S, H), F32),
    )(x1, moe, g2d, b2d)


def kernel(x, in_proj_w, in_proj_b, out_proj_w, out_proj_b,
           ln1_g, ln1_b, ln2_g, ln2_b, wg, w1, b1, w2, b2):
    x2d = x.reshape(S, H)
    wi_t = in_proj_w.T
    wo_t = out_proj_w.T
    bi2d = in_proj_b.reshape(1, 3 * H)
    bo2d = out_proj_b.reshape(1, H)
    wg_pad = jnp.pad(wg, ((0, 0), (0, EPAD - E)))

    qkv = _qkv(x2d, wi_t, bi2d)
    q3 = qkv[:, 0:H].reshape(S, NH, DH).transpose(1, 0, 2)
    k3 = qkv[:, H:2 * H].reshape(S, NH, DH).transpose(1, 0, 2)
    v3 = qkv[:, 2 * H:].reshape(S, NH, DH).transpose(1, 0, 2)
    o3 = _attention(q3, k3, v3)
    attn = o3.transpose(1, 0, 2).reshape(S, H)
    x1, lg = _ln1(attn, wo_t, bo2d, x2d,
                  ln1_g.reshape(1, H), ln1_b.reshape(1, H), wg_pad)

    s1, s2, g1, g2 = _router(lg)
    s1f = s1.reshape(S)
    s2f = s2.reshape(S)
    g1f = g1.reshape(S)
    g2f = g2.reshape(S)

    x1p = jnp.pad(x1, ((0, 8), (0, 0)))
    ein, gate_slot = _dispatch_sc(s1f, s2f, g1f, g2f, x1p)
    gate3d = gate_slot[:NSLOT].reshape(E, CAP, 1)

    eo = _ffn(ein, w1, b1, w2, b2, gate3d)
    eop = jnp.pad(eo, ((0, 8), (0, 0)))

    moe = _combine_sc(s1f, s2f, eop)
    out = _ln2(x1, moe, ln2_g.reshape(1, H), ln2_b.reshape(1, H))
    return out.reshape(1, S, H)


# speedup vs baseline: 1.3424x; 1.0980x over previous
import functools

import jax
import jax.numpy as jnp
from jax import lax
from jax.experimental import pallas as pl
from jax.experimental.pallas import tpu as pltpu
from jax.experimental.pallas import tpu_sc as plsc

F32 = jnp.float32
I32 = jnp.int32

S, H, NH, E, FF = 2048, 1024, 16, 8, 4096
DH = H // NH
CAP = int(1.25 * S * 2 / E)
NSLOT = E * CAP
PADROW = S
DUMMY = NSLOT
EPAD = 128

NC, NSUB = 2, 16
NW = NC * NSUB


def _qkv_body(x_ref, w_ref, b_ref, o_ref):
    o_ref[...] = (
        jnp.dot(x_ref[...], w_ref[...], preferred_element_type=F32) + b_ref[...]
    )


def _qkv(x2d, wi_t, bi2d):
    return pl.pallas_call(
        _qkv_body,
        grid=(8, 6),
        in_specs=[
            pl.BlockSpec((S // 8, H), lambda i, j: (i, 0)),
            pl.BlockSpec((H, 512), lambda i, j: (0, j)),
            pl.BlockSpec((1, 512), lambda i, j: (0, j)),
        ],
        out_specs=pl.BlockSpec((S // 8, 512), lambda i, j: (i, j)),
        out_shape=jax.ShapeDtypeStruct((S, 3 * H), F32),
    )(x2d, wi_t, bi2d)


def _attn_body(q_ref, k_ref, v_ref, o_ref):
    q = q_ref[0]
    k = k_ref[0]
    v = v_ref[0]
    s = lax.dot_general(
        q, k, (((1,), (1,)), ((), ())), preferred_element_type=F32
    ) * (1.0 / 8.0)
    kc = S // 2
    s1 = s[:, :kc]
    s2 = s[:, kc:]
    m1 = jnp.max(s1, axis=-1, keepdims=True)
    e1 = jnp.exp(s1 - m1)
    sum1 = jnp.sum(e1, axis=-1, keepdims=True)
    o1 = jnp.dot(e1, v[:kc], preferred_element_type=F32) * (1.0 / sum1)
    m2 = jnp.maximum(m1, jnp.max(s2, axis=-1, keepdims=True))
    corr = jnp.exp(jnp.where(m1 == m2, 0.0, m1 - m2))
    e2 = jnp.exp(s2 - m2)
    sum_sc = corr * sum1
    sum2 = sum_sc + jnp.sum(e2, axis=-1, keepdims=True)
    raw2 = jnp.dot(e2, v[kc:], preferred_element_type=F32) + sum_sc * o1
    o_ref[0] = raw2 * (1.0 / sum2)


def _attention(q3, k3, v3):
    qb = S // 8
    return pl.pallas_call(
        _attn_body,
        grid=(NH, 8),
        in_specs=[
            pl.BlockSpec((1, qb, DH), lambda h, i: (h, i, 0)),
            pl.BlockSpec((1, S, DH), lambda h, i: (h, 0, 0)),
            pl.BlockSpec((1, S, DH), lambda h, i: (h, 0, 0)),
        ],
        out_specs=pl.BlockSpec((1, qb, DH), lambda h, i: (h, i, 0)),
        out_shape=jax.ShapeDtypeStruct((NH, S, DH), F32),
    )(q3, k3, v3)


def _ln1_body(a_ref, wo_ref, bo_ref, x_ref, g_ref, b_ref, wg_ref, x1_ref, lg_ref):
    y = (
        jnp.dot(a_ref[...], wo_ref[...], preferred_element_type=F32)
        + bo_ref[...]
        + x_ref[...]
    )
    mu = jnp.mean(y, axis=-1, keepdims=True)
    yc = y - mu
    var = jnp.mean(yc * yc, axis=-1, keepdims=True)
    x1 = yc / jnp.sqrt(var + 1e-5) * g_ref[...] + b_ref[...]
    x1_ref[...] = x1
    lg_ref[...] = jnp.dot(x1, wg_ref[...], preferred_element_type=F32)


def _ln1(attn, wo_t, bo2d, x2d, g2d, b2d, wg_pad):
    rb = S // 8
    return pl.pallas_call(
        _ln1_body,
        grid=(8,),
        in_specs=[
            pl.BlockSpec((rb, H), lambda i: (i, 0)),
            pl.BlockSpec((H, H), lambda i: (0, 0)),
            pl.BlockSpec((1, H), lambda i: (0, 0)),
            pl.BlockSpec((rb, H), lambda i: (i, 0)),
            pl.BlockSpec((1, H), lambda i: (0, 0)),
            pl.BlockSpec((1, H), lambda i: (0, 0)),
            pl.BlockSpec((H, EPAD), lambda i: (0, 0)),
        ],
        out_specs=[
            pl.BlockSpec((rb, H), lambda i: (i, 0)),
            pl.BlockSpec((rb, EPAD), lambda i: (i, 0)),
        ],
        out_shape=[
            jax.ShapeDtypeStruct((S, H), F32),
            jax.ShapeDtypeStruct((S, EPAD), F32),
        ],
    )(attn, wo_t, bo2d, x2d, g2d, b2d, wg_pad)


_RB = 256
_NB = S // _RB


def _router_body(lg_ref, s1_ref, s2_ref, g1_ref, g2_ref,
                 m1_ref, m2_ref, p_ref, i1_ref, i2_ref, bs1_ref, bs2_ref):
    eiota = lax.broadcasted_iota(I32, (_RB, EPAD), 1)
    neg = jnp.float32(-1e30)
    for i in range(_NB):
        rows = pl.ds(i * _RB, _RB)
        logits = jnp.where(eiota < E, lg_ref[rows, :], neg)
        mx = jnp.max(logits, axis=-1, keepdims=True)
        ex = jnp.exp(logits - mx)
        probs = ex / jnp.sum(ex, axis=-1, keepdims=True)
        m1v = jnp.max(probs, axis=-1, keepdims=True)
        idx1 = jnp.min(jnp.where(probs == m1v, eiota, EPAD), axis=-1, keepdims=True)
        mask1 = (eiota == idx1).astype(F32)
        probs2 = probs * (1.0 - mask1)
        m2v = jnp.max(probs2, axis=-1, keepdims=True)
        idx2 = jnp.min(jnp.where(probs2 == m2v, eiota, EPAD), axis=-1, keepdims=True)
        mask2 = (eiota == idx2).astype(F32)
        m1_ref[rows, :] = mask1
        m2_ref[rows, :] = mask2
        p_ref[rows, :] = probs
        i1_ref[rows, :] = idx1
        i2_ref[rows, :] = idx2
        bs1_ref[pl.ds(i, 1), :] = jnp.sum(mask1, axis=0, keepdims=True)
        bs2_ref[pl.ds(i, 1), :] = jnp.sum(mask2, axis=0, keepdims=True)
    bs1 = bs1_ref[...]
    bs2 = bs2_ref[...]
    count1 = jnp.sum(bs1, axis=0, keepdims=True)
    tri = (
        lax.broadcasted_iota(I32, (_RB, _RB), 1)
        < lax.broadcasted_iota(I32, (_RB, _RB), 0)
    ).astype(F32)
    run1 = jnp.zeros((1, EPAD), F32)
    run2 = jnp.zeros((1, EPAD), F32)
    capf = jnp.float32(CAP)
    for i in range(_NB):
        rows = pl.ds(i * _RB, _RB)
        m1 = m1_ref[rows, :]
        m2 = m2_ref[rows, :]
        probs = p_ref[rows, :]
        loc1 = jnp.dot(tri, m1, preferred_element_type=F32) + run1
        loc2 = jnp.dot(tri, m2, preferred_element_type=F32) + run2 + count1
        run1 = run1 + bs1[i : i + 1, :]
        run2 = run2 + bs2[i : i + 1, :]
        m1c = m1 * (loc1 < capf).astype(F32)
        m2c = m2 * (loc2 < capf).astype(F32)
        g1 = jnp.sum(probs * m1c, axis=-1, keepdims=True)
        g2 = jnp.sum(probs * m2c, axis=-1, keepdims=True)
        den = g1 + g2 + 1e-9
        pos1 = jnp.sum(loc1 * m1c, axis=-1, keepdims=True).astype(I32)
        pos2 = jnp.sum(loc2 * m2c, axis=-1, keepdims=True).astype(I32)
        kept1 = jnp.sum(m1c, axis=-1, keepdims=True) > 0.0
        kept2 = jnp.sum(m2c, axis=-1, keepdims=True) > 0.0
        s1_ref[rows, :] = jnp.where(kept1, i1_ref[rows, :] * CAP + pos1, DUMMY)
        s2_ref[rows, :] = jnp.where(kept2, i2_ref[rows, :] * CAP + pos2, DUMMY)
        g1_ref[rows, :] = g1 / den
        g2_ref[rows, :] = g2 / den


def _router(lg):
    return pl.pallas_call(
        _router_body,
        in_specs=[pl.BlockSpec((S, EPAD), lambda: (0, 0))],
        out_specs=[
            pl.BlockSpec((S, 1), lambda: (0, 0)),
            pl.BlockSpec((S, 1), lambda: (0, 0)),
            pl.BlockSpec((S, 1), lambda: (0, 0)),
            pl.BlockSpec((S, 1), lambda: (0, 0)),
        ],
        out_shape=[
            jax.ShapeDtypeStruct((S, 1), I32),
            jax.ShapeDtypeStruct((S, 1), I32),
            jax.ShapeDtypeStruct((S, 1), F32),
            jax.ShapeDtypeStruct((S, 1), F32),
        ],
        scratch_shapes=[
            pltpu.VMEM((S, EPAD), F32),
            pltpu.VMEM((S, EPAD), F32),
            pltpu.VMEM((S, EPAD), F32),
            pltpu.VMEM((S, 1), I32),
            pltpu.VMEM((S, 1), I32),
            pltpu.VMEM((_NB, EPAD), F32),
            pltpu.VMEM((_NB, EPAD), F32),
        ],
    )(lg)


SLOT_PAD = 5136
ROWS_CH = 40
SL_PER_W = NSLOT // NW


def _disp_body(s1_hbm, s2_hbm, g1_hbm, g2_hbm, x1p_hbm,
               ein_hbm, gate_hbm,
               s1_v, s2_v, g1_v, g2_v, stok_v, gate_v, rows_v, sem):
    c = lax.axis_index("c")
    sid = lax.axis_index("s")
    wid = sid * NC + c
    pltpu.sync_copy(s1_hbm, s1_v)
    pltpu.sync_copy(s2_hbm, s2_v)
    pltpu.sync_copy(g1_hbm, g1_v)
    pltpu.sync_copy(g2_hbm, g2_v)

    def init_body(j, _):
        stok_v[pl.ds(j * 16, 16)] = jnp.full((16,), PADROW, I32)
        gate_v[pl.ds(j * 16, 16)] = jnp.zeros((16,), F32)
        return ()

    lax.fori_loop(0, SLOT_PAD // 16, init_body, ())

    def scat_body(i, _):
        tok = lax.broadcasted_iota(I32, (16,), 0) + i * 16
        sv1 = s1_v[pl.ds(i * 16, 16)]
        sv2 = s2_v[pl.ds(i * 16, 16)]
        plsc.store_scatter(stok_v, [sv1], tok)
        plsc.store_scatter(stok_v, [sv2], tok)
        plsc.store_scatter(gate_v, [sv1], g1_v[pl.ds(i * 16, 16)])
        plsc.store_scatter(gate_v, [sv2], g2_v[pl.ds(i * 16, 16)])
        return ()

    lax.fori_loop(0, S // 16, scat_body, ())

    @pl.when(wid == 0)
    def _():
        pltpu.sync_copy(gate_v.at[pl.ds(0, NSLOT + 8)], gate_hbm)

    for ch in range(SL_PER_W // ROWS_CH):
        base = wid * SL_PER_W + ch * ROWS_CH
        cp = pltpu.async_copy(
            x1p_hbm.at[stok_v.at[pl.ds(base, ROWS_CH)]], rows_v, sem
        )
        cp.wait()
        pltpu.sync_copy(rows_v, ein_hbm.at[pl.ds(base, ROWS_CH)])


def _dispatch_sc(s1, s2, g1, g2, x1p):
    f = pl.kernel(
        _disp_body,
        out_type=[
            jax.ShapeDtypeStruct((NSLOT, H), F32),
            jax.ShapeDtypeStruct((NSLOT + 8,), F32),
        ],
        mesh=plsc.VectorSubcoreMesh(core_axis_name="c", subcore_axis_name="s"),
        compiler_params=pltpu.CompilerParams(needs_layout_passes=False),
        scratch_types=[
            pltpu.VMEM((S,), I32),
            pltpu.VMEM((S,), I32),
            pltpu.VMEM((S,), F32),
            pltpu.VMEM((S,), F32),
            pltpu.VMEM((SLOT_PAD,), I32),
            pltpu.VMEM((SLOT_PAD,), F32),
            pltpu.VMEM((ROWS_CH, H), F32),
            pltpu.SemaphoreType.DMA,
        ],
    )
    return f(s1, s2, g1, g2, x1p)


_FB = 1024
_SQRT1_2 = 0.7071067811865476


def _erf(x):
    a1, a2, a3, a4, a5 = (
        0.254829592, -0.284496736, 1.421413741, -1.453152027, 1.061405429,
    )
    p = 0.3275911
    ax = jnp.abs(x)
    t = 1.0 / (1.0 + p * ax)
    y = 1.0 - ((((a5 * t + a4) * t + a3) * t + a2) * t + a1) * t * jnp.exp(-ax * ax)
    return jnp.sign(x) * y


def _gelu(x):
    return 0.5 * x * (1.0 + _erf(x * _SQRT1_2))


def _ffn_body(xin_ref, w1_ref, b1_ref, w2_ref, b2_ref, g_ref, o_ref):
    f = pl.program_id(1)
    nf = pl.num_programs(1)
    h = _gelu(
        jnp.dot(xin_ref[...], w1_ref[0], preferred_element_type=F32) + b1_ref[0]
    )
    part = jnp.dot(h, w2_ref[0], preferred_element_type=F32)

    @pl.when(f == 0)
    def _():
        o_ref[...] = part

    @pl.when(f > 0)
    def _():
        o_ref[...] += part

    @pl.when(f == nf - 1)
    def _():
        eo = (o_ref[...] + b2_ref[0]).astype(jnp.bfloat16).astype(F32)
        gq = g_ref[0].astype(jnp.bfloat16).astype(F32)
        o_ref[...] = eo * gq


def _ffn(ein, w1, b1, w2, b2, gate3d):
    return pl.pallas_call(
        _ffn_body,
        grid=(E, FF // _FB),
        in_specs=[
            pl.BlockSpec((CAP, H), lambda e, f: (e, 0)),
            pl.BlockSpec((1, H, _FB), lambda e, f: (e, 0, f)),
            pl.BlockSpec((1, 1, _FB), lambda e, f: (e, 0, f)),
            pl.BlockSpec((1, _FB, H), lambda e, f: (e, f, 0)),
            pl.BlockSpec((1, 1, H), lambda e, f: (e, 0, 0)),
            pl.BlockSpec((1, CAP, 1), lambda e, f: (e, 0, 0)),
        ],
        out_specs=pl.BlockSpec((CAP, H), lambda e, f: (e, 0)),
        out_shape=jax.ShapeDtypeStruct((NSLOT, H), F32),
    )(ein, w1, b1.reshape(E, 1, FF), w2, b2.reshape(E, 1, H), gate3d)


TPB = S // NW
CHT = 32


def _comb_body(s1_hbm, s2_hbm, eop_hbm, out_hbm, i1_v, i2_v, r1_v, r2_v, sem1, sem2):
    c = lax.axis_index("c")
    sid = lax.axis_index("s")
    wid = sid * NC + c
    t0 = wid * TPB
    pltpu.sync_copy(s1_hbm.at[pl.ds(t0, TPB)], i1_v)
    pltpu.sync_copy(s2_hbm.at[pl.ds(t0, TPB)], i2_v)
    for ch in range(TPB // CHT):
        cb = ch * CHT
        cp1 = pltpu.async_copy(eop_hbm.at[i1_v.at[pl.ds(cb, CHT)]], r1_v, sem1)
        cp2 = pltpu.async_copy(eop_hbm.at[i2_v.at[pl.ds(cb, CHT)]], r2_v, sem2)
        cp1.wait()
        cp2.wait()

        def row_body(r, _):
            for cc in range(H // 16):
                sl = pl.ds(cc * 16, 16)
                r1_v[r, sl] = r1_v[r, sl] + r2_v[r, sl]
            return ()

        lax.fori_loop(0, CHT, row_body, ())
        pltpu.sync_copy(r1_v, out_hbm.at[pl.ds(t0 + cb, CHT)])


def _combine_sc(s1, s2, eop):
    f = pl.kernel(
        _comb_body,
        out_type=jax.ShapeDtypeStruct((S, H), F32),
        mesh=plsc.VectorSubcoreMesh(core_axis_name="c", subcore_axis_name="s"),
        compiler_params=pltpu.CompilerParams(needs_layout_passes=False),
        scratch_types=[
            pltpu.VMEM((TPB,), I32),
            pltpu.VMEM((TPB,), I32),
            pltpu.VMEM((CHT, H), F32),
            pltpu.VMEM((CHT, H), F32),
            pltpu.SemaphoreType.DMA,
            pltpu.SemaphoreType.DMA,
        ],
    )
    return f(s1, s2, eop)


def _ln2_body(x1_ref, m_ref, g_ref, b_ref, o_ref):
    y = x1_ref[...] + m_ref[...]
    mu = jnp.mean(y, axis=-1, keepdims=True)
    yc = y - mu
    var = jnp.mean(yc * yc, axis=-1, keepdims=True)
    o_ref[...] = yc * lax.rsqrt(var + 1e-5) * g_ref[...] + b_ref[...]


def _ln2(x1, moe, g2d, b2d):
    rb = S // 8
    return pl.pallas_call(
        _ln2_body,
        grid=(8,),
        in_specs=[
            pl.BlockSpec((rb, H), lambda i: (i, 0)),
            pl.BlockSpec((rb, H), lambda i: (i, 0)),
            pl.BlockSpec((1, H), lambda i: (0, 0)),
            pl.BlockSpec((1, H), lambda i: (0, 0)),
        ],
        out_specs=pl.BlockSpec((rb, H), lambda i: (i, 0)),
        out_shape=jax.ShapeDtypeStruct((S, H), F32),
    )(x1, moe, g2d, b2d)


def kernel(x, in_proj_w, in_proj_b, out_proj_w, out_proj_b,
           ln1_g, ln1_b, ln2_g, ln2_b, wg, w1, b1, w2, b2):
    x2d = x.reshape(S, H)
    wi_t = in_proj_w.T
    wo_t = out_proj_w.T
    bi2d = in_proj_b.reshape(1, 3 * H)
    bo2d = out_proj_b.reshape(1, H)
    wg_pad = jnp.pad(wg, ((0, 0), (0, EPAD - E)))

    qkv = _qkv(x2d, wi_t, bi2d)
    q3 = qkv[:, 0:H].reshape(S, NH, DH).transpose(1, 0, 2)
    k3 = qkv[:, H:2 * H].reshape(S, NH, DH).transpose(1, 0, 2)
    v3 = qkv[:, 2 * H:].reshape(S, NH, DH).transpose(1, 0, 2)
    o3 = _attention(q3, k3, v3)
    attn = o3.transpose(1, 0, 2).reshape(S, H)
    x1, lg = _ln1(attn, wo_t, bo2d, x2d,
                  ln1_g.reshape(1, H), ln1_b.reshape(1, H), wg_pad)

    s1, s2, g1, g2 = _router(lg)
    s1f = s1.reshape(S)
    s2f = s2.reshape(S)
    g1f = g1.reshape(S)
    g2f = g2.reshape(S)

    x1p = jnp.pad(x1, ((0, 8), (0, 0)))
    ein, gate_slot = _dispatch_sc(s1f, s2f, g1f, g2f, x1p)
    gate3d = gate_slot[:NSLOT].reshape(E, CAP, 1)

    eo = _ffn(ein, w1, b1, w2, b2, gate3d)
    eop = jnp.pad(eo, ((0, 8), (0, 0)))

    moe = _combine_sc(s1f, s2f, eop)
    out = _ln2(x1, moe, ln2_g.reshape(1, H), ln2_b.reshape(1, H))
    return out.reshape(1, S, H)
